# Initial kernel scaffold; baseline (speedup 1.0000x reference)
import jax
import jax.numpy as jnp
from jax import lax
from jax.experimental import pallas as pl
from jax.experimental.pallas import tpu as pltpu
from jax.experimental.pallas import tpu_sc as plsc

NN = 50000
EE = 800000
BB = 64
FD = 64

NSC = 2
NTL = 16

NH = 25088
NPAD = NH * NSC
TBL = NH + 128
RT = NH // NTL
ZT = TBL // NTL

KC = 128
SUB = 8
OUTER = 49
EPT = OUTER * SUB * KC
EPAD = EPT * NTL
ER = EPAD // KC
ERT = ER // NTL

RC = 112

_mesh = plsc.VectorSubcoreMesh(core_axis_name="c", subcore_axis_name="s")


def _zero_vmem2(buf, rows):
    z = jnp.zeros((16,), jnp.float32)

    def body(k, _):
        for q in range(FD // 16):
            buf[k, pl.ds(q * 16, 16)] = z
        return _

    lax.fori_loop(0, rows, body, None)


def _zero_shared2(zbuf, sh, s):
    base = s * ZT
    nfull = ZT // KC
    rem = ZT - nfull * KC
    for i in range(nfull):
        pltpu.sync_copy(zbuf, sh.at[pl.ds(base + i * KC, KC)])
    pltpu.sync_copy(zbuf.at[pl.ds(0, rem)], sh.at[pl.ds(base + nfull * KC, rem)])


def _local_col_idx(colbuf, idxbuf, c):
    base = c * NH
    for j in range(SUB):
        for q in range(KC // 16):
            sl = pl.ds(q * 16, 16)
            v = colbuf[j, sl] - base
            ok = (v >= 0) & (v < NH)
            idxbuf[j, sl] = jnp.where(ok, v, NH)


def _edge_scatter_pass(y_hbm, row_hbm, col_hbm, w_hbm, acc_sh,
                       rowi, coli, wbuf, idxb, rbuf, sem, c, s):
    def outer(ch, _):
        r0 = s * ERT + ch * SUB
        pltpu.sync_copy(row_hbm.at[pl.ds(r0, SUB)], rowi)
        pltpu.sync_copy(col_hbm.at[pl.ds(r0, SUB)], coli)
        pltpu.sync_copy(w_hbm.at[pl.ds(r0, SUB)], wbuf)
        _local_col_idx(coli, idxb, c)
        for j in range(SUB):
            pltpu.async_copy(y_hbm.at[rowi.at[j]], rbuf, sem).wait()

            def mul(k, _):
                for u in range(4):
                    kk = k * 4 + u
                    wk = wbuf[j, kk]
                    for q in range(FD // 16):
                        sl = pl.ds(q * 16, 16)
                        rbuf[kk, sl] = rbuf[kk, sl] * wk
                return _

            lax.fori_loop(0, KC // 4, mul, None)
            pltpu.sync_copy(rbuf, acc_sh.at[idxb.at[j]], add=True)
        return _

    lax.fori_loop(0, OUTER, outer, None)


def _deg_body(col_hbm, w_hbm, deg_hbm, deg_sh, coli, wbuf, idxb, zbuf):
    c = lax.axis_index("c")
    s = lax.axis_index("s")
    z = jnp.zeros((16,), jnp.float32)
    for q in range(KC // 16):
        zbuf[pl.ds(q * 16, 16)] = z
    base = s * ZT
    nfull = ZT // KC
    rem = ZT - nfull * KC
    for i in range(nfull):
        pltpu.sync_copy(zbuf, deg_sh.at[pl.ds(base + i * KC, KC)])
    pltpu.sync_copy(zbuf.at[pl.ds(0, rem)], deg_sh.at[pl.ds(base + nfull * KC, rem)])
    plsc.subcore_barrier()

    def outer(ch, _):
        r0 = s * ERT + ch * SUB
        pltpu.sync_copy(col_hbm.at[pl.ds(r0, SUB)], coli)
        pltpu.sync_copy(w_hbm.at[pl.ds(r0, SUB)], wbuf)
        _local_col_idx(coli, idxb, c)
        for j in range(SUB):
            pltpu.sync_copy(wbuf.at[j], deg_sh.at[idxb.at[j]], add=True)
        return _

    lax.fori_loop(0, OUTER, outer, None)
    plsc.subcore_barrier()
    pltpu.sync_copy(deg_sh.at[pl.ds(s * RT, RT)],
                    deg_hbm.at[pl.ds(c * NH + s * RT, RT)])


_deg_call = pl.kernel(
    _deg_body,
    out_type=jax.ShapeDtypeStruct((NPAD,), jnp.float32),
    mesh=_mesh,
    scratch_types=[
        pltpu.VMEM_SHARED((TBL,), jnp.float32),
        pltpu.VMEM((SUB, KC), jnp.int32),
        pltpu.VMEM((SUB, KC), jnp.float32),
        pltpu.VMEM((SUB, KC), jnp.int32),
        pltpu.VMEM((KC,), jnp.float32),
    ],
)


def _conv_body(y_hbm, row_hbm, col_hbm, w_hbm, acc_hbm,
               acc_sh, rowi, coli, wbuf, idxb, rbuf, zbuf, sem):
    c = lax.axis_index("c")
    s = lax.axis_index("s")
    _zero_vmem2(zbuf, KC)
    _zero_shared2(zbuf, acc_sh, s)
    plsc.subcore_barrier()
    _edge_scatter_pass(y_hbm, row_hbm, col_hbm, w_hbm, acc_sh,
                       rowi, coli, wbuf, idxb, rbuf, sem, c, s)
    plsc.subcore_barrier()
    pltpu.sync_copy(acc_sh.at[pl.ds(s * RT, RT)],
                    acc_hbm.at[pl.ds(c * NH + s * RT, RT)])


_conv_call = pl.kernel(
    _conv_body,
    out_type=jax.ShapeDtypeStruct((NPAD, FD), jnp.float32),
    mesh=_mesh,
    scratch_types=[
        pltpu.VMEM_SHARED((TBL, FD), jnp.float32),
        pltpu.VMEM((SUB, KC), jnp.int32),
        pltpu.VMEM((SUB, KC), jnp.int32),
        pltpu.VMEM((SUB, KC), jnp.float32),
        pltpu.VMEM((SUB, KC), jnp.int32),
        pltpu.VMEM((KC, FD), jnp.float32),
        pltpu.VMEM((KC, FD), jnp.float32),
        pltpu.SemaphoreType.DMA,
    ],
)


def _conv_segmax_body(y_hbm, row_hbm, col_hbm, w_hbm, dis_hbm, bat_hbm, b2_hbm,
                      part_hbm, acc_sh, rowi, coli, wbuf, idxb, rbuf, zbuf,
                      abuf, ybuf, dbuf, bbuf, b2buf, segbuf, sem):
    c = lax.axis_index("c")
    s = lax.axis_index("s")
    _zero_vmem2(zbuf, KC)
    _zero_shared2(zbuf, acc_sh, s)
    plsc.subcore_barrier()
    _edge_scatter_pass(y_hbm, row_hbm, col_hbm, w_hbm, acc_sh,
                       rowi, coli, wbuf, idxb, rbuf, sem, c, s)
    plsc.subcore_barrier()

    ninf = jnp.full((16,), -jnp.inf, jnp.float32)

    def seg_init(k, _):
        for q in range(FD // 16):
            segbuf[k, pl.ds(q * 16, 16)] = ninf
        return _

    lax.fori_loop(0, BB, seg_init, None)

    pltpu.sync_copy(b2_hbm, b2buf)
    gstart = c * NH + s * RT
    nvalid = jnp.clip(NN - gstart, 0, RT)

    def seg_chunk(i, _):
        l0 = s * RT + i * RC
        g0 = gstart + i * RC
        pltpu.sync_copy(acc_sh.at[pl.ds(l0, RC)], abuf)
        pltpu.sync_copy(y_hbm.at[pl.ds(g0, RC)], ybuf)
        pltpu.sync_copy(dis_hbm.at[pl.ds(g0, RC)], dbuf)
        pltpu.sync_copy(bat_hbm.at[pl.ds(g0, RC)], bbuf)
        cnt = jnp.clip(nvalid - i * RC, 0, RC)

        def seg_row(k, _):
            dk = dbuf[k]
            bid = bbuf[k]
            for q in range(FD // 16):
                sl = pl.ds(q * 16, 16)
                v = dk * (abuf[k, sl] + ybuf[k, sl]) + b2buf[sl]
                segbuf[bid, sl] = jnp.maximum(segbuf[bid, sl], v)
            return _

        lax.fori_loop(0, cnt, seg_row, None)
        return _

    lax.fori_loop(0, RT // RC, seg_chunk, None)
    pltpu.sync_copy(segbuf, part_hbm.at[c, s])


_conv_segmax_call = pl.kernel(
    _conv_segmax_body,
    out_type=jax.ShapeDtypeStruct((NSC, NTL, BB, FD), jnp.float32),
    mesh=_mesh,
    scratch_types=[
        pltpu.VMEM_SHARED((TBL, FD), jnp.float32),
        pltpu.VMEM((SUB, KC), jnp.int32),
        pltpu.VMEM((SUB, KC), jnp.int32),
        pltpu.VMEM((SUB, KC), jnp.float32),
        pltpu.VMEM((SUB, KC), jnp.int32),
        pltpu.VMEM((KC, FD), jnp.float32),
        pltpu.VMEM((KC, FD), jnp.float32),
        pltpu.VMEM((RC, FD), jnp.float32),
        pltpu.VMEM((RC, FD), jnp.float32),
        pltpu.VMEM((RC,), jnp.float32),
        pltpu.VMEM((RC,), jnp.int32),
        pltpu.VMEM((FD,), jnp.float32),
        pltpu.VMEM((BB, FD), jnp.float32),
        pltpu.SemaphoreType.DMA,
    ],
)


_NB = NPAD // 128


def _k2_body(pos_ref, deg_ref, w1_ref, dis_ref, y1_ref):
    dis = lax.rsqrt(deg_ref[...] + 1.0)
    dis_ref[...] = dis
    xl = jnp.dot(pos_ref[...], w1_ref[...], preferred_element_type=jnp.float32)
    y1_ref[...] = xl * dis


def _k2(posp, deg2d, W1):
    return pl.pallas_call(
        _k2_body,
        grid=(_NB,),
        in_specs=[
            pl.BlockSpec((128, 3), lambda i: (i, 0)),
            pl.BlockSpec((128, 1), lambda i: (i, 0)),
            pl.BlockSpec((3, FD), lambda i: (0, 0)),
        ],
        out_specs=[
            pl.BlockSpec((128, 1), lambda i: (i, 0)),
            pl.BlockSpec((128, FD), lambda i: (i, 0)),
        ],
        out_shape=[
            jax.ShapeDtypeStruct((NPAD, 1), jnp.float32),
            jax.ShapeDtypeStruct((NPAD, FD), jnp.float32),
        ],
    )(posp, deg2d, W1)


def _k4_body(acc_ref, y1_ref, dis_ref, w2_ref, b1_ref, y2_ref):
    dis = dis_ref[...]
    x1 = jnp.maximum(dis * (acc_ref[...] + y1_ref[...]) + b1_ref[...], 0.0)
    y2_ref[...] = jnp.dot(x1, w2_ref[...], preferred_element_type=jnp.float32) * dis


def _k4(acc1, y1, dis2d, W2, b1):
    return pl.pallas_call(
        _k4_body,
        grid=(_NB,),
        in_specs=[
            pl.BlockSpec((128, FD), lambda i: (i, 0)),
            pl.BlockSpec((128, FD), lambda i: (i, 0)),
            pl.BlockSpec((128, 1), lambda i: (i, 0)),
            pl.BlockSpec((FD, FD), lambda i: (0, 0)),
            pl.BlockSpec((1, FD), lambda i: (0, 0)),
        ],
        out_specs=pl.BlockSpec((128, FD), lambda i: (i, 0)),
        out_shape=jax.ShapeDtypeStruct((NPAD, FD), jnp.float32),
    )(acc1, y1, dis2d, W2, b1)


def _k6_body(part_ref, wm1_ref, bm1_ref, g1_ref, be1_ref,
             wm2_ref, bm2_ref, g2_ref, be2_ref, wm3_ref, bm3_ref, out_ref):
    x = jnp.max(part_ref[...], axis=(0, 1))

    def bn_relu(h, g, be):
        m = jnp.mean(h, axis=0, keepdims=True)
        v = jnp.mean((h - m) ** 2, axis=0, keepdims=True)
        return jnp.maximum(g * (h - m) / jnp.sqrt(v + 1e-5) + be, 0.0)

    h = jnp.dot(x, wm1_ref[...], preferred_element_type=jnp.float32) + bm1_ref[...]
    h = bn_relu(h, g1_ref[...], be1_ref[...])
    h = jnp.dot(h, wm2_ref[...], preferred_element_type=jnp.float32) + bm2_ref[...]
    h = bn_relu(h, g2_ref[...], be2_ref[...])
    out_ref[...] = jnp.dot(h, wm3_ref[...], preferred_element_type=jnp.float32) + bm3_ref[...]


def _k6(part, Wm1, bm1, g1, be1, Wm2, bm2, g2, be2, Wm3, bm3):
    return pl.pallas_call(
        _k6_body,
        out_shape=jax.ShapeDtypeStruct((BB, 10), jnp.float32),
    )(part, Wm1, bm1, g1, be1, Wm2, bm2, g2, be2, Wm3, bm3)


def kernel(pos, edge_index, edge_attr, batch,
           W1, b1, W2, b2, Wm1, bm1, g1, be1, Wm2, bm2, g2, be2, Wm3, bm3):
    pe = EPAD - EE
    rowp = jnp.concatenate([edge_index[0], jnp.zeros((pe,), jnp.int32)]).reshape(ER, KC)
    colp = jnp.concatenate([edge_index[1], jnp.zeros((pe,), jnp.int32)]).reshape(ER, KC)
    wp = jnp.concatenate([edge_attr, jnp.zeros((pe,), jnp.float32)]).reshape(ER, KC)
    posp = jnp.concatenate([pos, jnp.zeros((NPAD - NN, 3), jnp.float32)])
    batp = jnp.concatenate([batch, jnp.full((NPAD - NN,), BB - 1, jnp.int32)])

    deg = _deg_call(colp, wp)
    dis2d, y1 = _k2(posp, deg.reshape(NPAD, 1), W1)
    acc1 = _conv_call(y1, rowp, colp, wp)
    y2 = _k4(acc1, y1, dis2d, W2, b1.reshape(1, FD))
    part = _conv_segmax_call(y2, rowp, colp, wp, dis2d.reshape(NPAD), batp, b2)
    return _k6(part, Wm1, bm1.reshape(1, FD), g1.reshape(1, FD), be1.reshape(1, FD),
               Wm2, bm2.reshape(1, FD), g2.reshape(1, FD), be2.reshape(1, FD),
               Wm3, bm3.reshape(1, 10))


# speedup vs baseline: 7.1208x; 7.1208x over previous
import jax
import jax.numpy as jnp
from jax import lax
from jax.experimental import pallas as pl
from jax.experimental.pallas import tpu as pltpu
from jax.experimental.pallas import tpu_sc as plsc

NN = 50000
EE = 800000
BB = 64
FD = 64

NSC = 2
NTL = 16

NH = 25088
NPAD = NH * NSC
TBL = NH + 128
RT = NH // NTL
ZT = TBL // NTL

KC = 128
SUB = 8
OUTER = 49
EPT = OUTER * SUB * KC
EPAD = EPT * NTL
ER = EPAD // KC
ERT = ER // NTL

RC = 112

_mesh = plsc.VectorSubcoreMesh(core_axis_name="c", subcore_axis_name="s")
_sc_params = pltpu.CompilerParams(use_tc_tiling_on_sc=False)


def _zero_vmem2(buf, rows):
    z = jnp.zeros((16,), jnp.float32)

    def body(k, _):
        for q in range(FD // 16):
            buf[k, pl.ds(q * 16, 16)] = z
        return _

    lax.fori_loop(0, rows, body, None)


def _zero_shared2(zbuf, sh, s):
    base = s * ZT
    nfull = ZT // KC
    rem = ZT - nfull * KC
    for i in range(nfull):
        pltpu.sync_copy(zbuf, sh.at[pl.ds(base + i * KC, KC)])
    pltpu.sync_copy(zbuf.at[pl.ds(0, rem)], sh.at[pl.ds(base + nfull * KC, rem)])


def _local_col_idx(colbuf, idxbuf, c):
    base = c * NH
    for j in range(SUB):
        for q in range(KC // 16):
            sl = pl.ds(q * 16, 16)
            v = colbuf[j, sl] - base
            ok = (v >= 0) & (v < NH)
            idxbuf[j, sl] = jnp.where(ok, v, NH)


def _edge_scatter_pass(y_hbm, row_hbm, col_hbm, w_hbm, acc_sh,
                       rowi, coli, wbuf, idxb, rbuf, sem, c, s):
    def outer(ch, _):
        r0 = s * ERT + ch * SUB
        pltpu.sync_copy(row_hbm.at[pl.ds(r0, SUB)], rowi)
        pltpu.sync_copy(col_hbm.at[pl.ds(r0, SUB)], coli)
        pltpu.sync_copy(w_hbm.at[pl.ds(r0, SUB)], wbuf)
        _local_col_idx(coli, idxb, c)
        for j in range(SUB):
            pltpu.async_copy(y_hbm.at[rowi.at[j]], rbuf, sem).wait()

            def mul(g, _):
                wv = wbuf[j, pl.ds(g * 16, 16)]
                kb = g * 16
                for u in range(16):
                    wk = wv[u]
                    for q in range(FD // 16):
                        sl = pl.ds(q * 16, 16)
                        rbuf[kb + u, sl] = rbuf[kb + u, sl] * wk
                return _

            lax.fori_loop(0, KC // 16, mul, None)
            pltpu.sync_copy(rbuf, acc_sh.at[idxb.at[j]], add=True)
        return _

    lax.fori_loop(0, OUTER, outer, None)


def _deg_body(col_hbm, w_hbm, deg_hbm, deg_sh, coli, wbuf, idxb, zbuf):
    c = lax.axis_index("c")
    s = lax.axis_index("s")
    z = jnp.zeros((16,), jnp.float32)
    for q in range(KC // 16):
        zbuf[pl.ds(q * 16, 16)] = z
    base = s * ZT
    nfull = ZT // KC
    rem = ZT - nfull * KC
    for i in range(nfull):
        pltpu.sync_copy(zbuf, deg_sh.at[pl.ds(base + i * KC, KC)])
    pltpu.sync_copy(zbuf.at[pl.ds(0, rem)], deg_sh.at[pl.ds(base + nfull * KC, rem)])
    plsc.subcore_barrier()

    def outer(ch, _):
        r0 = s * ERT + ch * SUB
        pltpu.sync_copy(col_hbm.at[pl.ds(r0, SUB)], coli)
        pltpu.sync_copy(w_hbm.at[pl.ds(r0, SUB)], wbuf)
        _local_col_idx(coli, idxb, c)
        for j in range(SUB):
            pltpu.sync_copy(wbuf.at[j], deg_sh.at[idxb.at[j]], add=True)
        return _

    lax.fori_loop(0, OUTER, outer, None)
    plsc.subcore_barrier()
    nfo = RT // KC
    tail = RT - nfo * KC
    for i in range(nfo):
        pltpu.sync_copy(deg_sh.at[pl.ds(s * RT + i * KC, KC)], zbuf)
        pltpu.sync_copy(zbuf, deg_hbm.at[pl.ds(c * NH + s * RT + i * KC, KC)])
    pltpu.sync_copy(deg_sh.at[pl.ds(s * RT + nfo * KC, tail)], zbuf.at[pl.ds(0, tail)])
    pltpu.sync_copy(zbuf.at[pl.ds(0, tail)],
                    deg_hbm.at[pl.ds(c * NH + s * RT + nfo * KC, tail)])


_deg_call = pl.kernel(
    _deg_body,
    out_type=jax.ShapeDtypeStruct((NPAD,), jnp.float32),
    mesh=_mesh,
    compiler_params=_sc_params,
    scratch_types=[
        pltpu.VMEM_SHARED((TBL,), jnp.float32),
        pltpu.VMEM((SUB, KC), jnp.int32),
        pltpu.VMEM((SUB, KC), jnp.float32),
        pltpu.VMEM((SUB, KC), jnp.int32),
        pltpu.VMEM((KC,), jnp.float32),
    ],
)


def _conv_body(y_hbm, row_hbm, col_hbm, w_hbm, acc_hbm,
               acc_sh, rowi, coli, wbuf, idxb, rbuf, zbuf, sem):
    c = lax.axis_index("c")
    s = lax.axis_index("s")
    _zero_vmem2(zbuf, KC)
    _zero_shared2(zbuf, acc_sh, s)
    plsc.subcore_barrier()
    _edge_scatter_pass(y_hbm, row_hbm, col_hbm, w_hbm, acc_sh,
                       rowi, coli, wbuf, idxb, rbuf, sem, c, s)
    plsc.subcore_barrier()
    pltpu.sync_copy(acc_sh.at[pl.ds(s * RT, RT)],
                    acc_hbm.at[pl.ds(c * NH + s * RT, RT)])


_conv_call = pl.kernel(
    _conv_body,
    out_type=jax.ShapeDtypeStruct((NPAD, FD), jnp.float32),
    mesh=_mesh,
    compiler_params=_sc_params,
    scratch_types=[
        pltpu.VMEM_SHARED((TBL, FD), jnp.float32),
        pltpu.VMEM((SUB, KC), jnp.int32),
        pltpu.VMEM((SUB, KC), jnp.int32),
        pltpu.VMEM((SUB, KC), jnp.float32),
        pltpu.VMEM((SUB, KC), jnp.int32),
        pltpu.VMEM((KC, FD), jnp.float32),
        pltpu.VMEM((KC, FD), jnp.float32),
        pltpu.SemaphoreType.DMA,
    ],
)


def _conv_segmax_body(y_hbm, row_hbm, col_hbm, w_hbm, dis_hbm, bat_hbm, b2_hbm,
                      part_hbm, acc_sh, rowi, coli, wbuf, idxb, rbuf, zbuf,
                      dbuf, bbuf, b2buf, segbuf, sem):
    c = lax.axis_index("c")
    s = lax.axis_index("s")
    _zero_vmem2(zbuf, KC)
    _zero_shared2(zbuf, acc_sh, s)
    plsc.subcore_barrier()
    _edge_scatter_pass(y_hbm, row_hbm, col_hbm, w_hbm, acc_sh,
                       rowi, coli, wbuf, idxb, rbuf, sem, c, s)
    plsc.subcore_barrier()

    ninf = jnp.full((16,), -jnp.inf, jnp.float32)

    def seg_init(k, _):
        for q in range(FD // 16):
            segbuf[k, pl.ds(q * 16, 16)] = ninf
        return _

    lax.fori_loop(0, BB, seg_init, None)

    pltpu.sync_copy(b2_hbm, b2buf)
    gstart = c * NH + s * RT

    def seg_chunk(i, _):
        l0 = s * RT + i * RC
        g0 = gstart + i * RC
        abuf = zbuf.at[pl.ds(0, RC)]
        ybuf = rbuf.at[pl.ds(0, RC)]
        pltpu.sync_copy(acc_sh.at[pl.ds(l0, RC)], abuf)
        pltpu.sync_copy(y_hbm.at[pl.ds(g0, RC)], ybuf)
        pltpu.sync_copy(dis_hbm.at[pl.ds(g0, RC)], dbuf)
        pltpu.sync_copy(bat_hbm.at[pl.ds(g0, RC)], bbuf)

        def seg_grp(g, _):
            dv = dbuf[pl.ds(g * 16, 16)]
            bv = bbuf[pl.ds(g * 16, 16)]
            kb = g * 16
            gbase = g0 + kb
            for u in range(16):
                dk = dv[u]
                bid = bv[u]
                pen = jnp.where(gbase + u < NN, 0.0, -jnp.inf).astype(jnp.float32)
                for q in range(FD // 16):
                    sl = pl.ds(q * 16, 16)
                    v = dk * (zbuf[kb + u, sl] + rbuf[kb + u, sl]) + b2buf[sl] + pen
                    segbuf[bid, sl] = jnp.maximum(segbuf[bid, sl], v)
            return _

        lax.fori_loop(0, RC // 16, seg_grp, None)
        return _

    lax.fori_loop(0, RT // RC, seg_chunk, None)
    pltpu.sync_copy(segbuf, part_hbm.at[c, s])


_conv_segmax_call = pl.kernel(
    _conv_segmax_body,
    out_type=jax.ShapeDtypeStruct((NSC, NTL, BB, FD), jnp.float32),
    mesh=_mesh,
    compiler_params=_sc_params,
    scratch_types=[
        pltpu.VMEM_SHARED((TBL, FD), jnp.float32),
        pltpu.VMEM((SUB, KC), jnp.int32),
        pltpu.VMEM((SUB, KC), jnp.int32),
        pltpu.VMEM((SUB, KC), jnp.float32),
        pltpu.VMEM((SUB, KC), jnp.int32),
        pltpu.VMEM((KC, FD), jnp.float32),
        pltpu.VMEM((KC, FD), jnp.float32),
        pltpu.VMEM((RC,), jnp.float32),
        pltpu.VMEM((RC,), jnp.int32),
        pltpu.VMEM((FD,), jnp.float32),
        pltpu.VMEM((BB, FD), jnp.float32),
        pltpu.SemaphoreType.DMA,
    ],
)


_NB = NPAD // 128


def _k2_body(pos_ref, deg_ref, w1_ref, dis_ref, y1_ref):
    dis = lax.rsqrt(deg_ref[...] + 1.0)
    dis_ref[...] = dis
    xl = jnp.dot(pos_ref[...], w1_ref[...], preferred_element_type=jnp.float32)
    y1_ref[...] = xl * dis


def _k2(posp, deg2d, W1):
    return pl.pallas_call(
        _k2_body,
        grid=(_NB,),
        in_specs=[
            pl.BlockSpec((128, 3), lambda i: (i, 0)),
            pl.BlockSpec((128, 1), lambda i: (i, 0)),
            pl.BlockSpec((3, FD), lambda i: (0, 0)),
        ],
        out_specs=[
            pl.BlockSpec((128, 1), lambda i: (i, 0)),
            pl.BlockSpec((128, FD), lambda i: (i, 0)),
        ],
        out_shape=[
            jax.ShapeDtypeStruct((NPAD, 1), jnp.float32),
            jax.ShapeDtypeStruct((NPAD, FD), jnp.float32),
        ],
    )(posp, deg2d, W1)


def _k4_body(acc_ref, y1_ref, dis_ref, w2_ref, b1_ref, y2_ref):
    dis = dis_ref[...]
    x1 = jnp.maximum(dis * (acc_ref[...] + y1_ref[...]) + b1_ref[...], 0.0)
    y2_ref[...] = jnp.dot(x1, w2_ref[...], preferred_element_type=jnp.float32) * dis


def _k4(acc1, y1, dis2d, W2, b1):
    return pl.pallas_call(
        _k4_body,
        grid=(_NB,),
        in_specs=[
            pl.BlockSpec((128, FD), lambda i: (i, 0)),
            pl.BlockSpec((128, FD), lambda i: (i, 0)),
            pl.BlockSpec((128, 1), lambda i: (i, 0)),
            pl.BlockSpec((FD, FD), lambda i: (0, 0)),
            pl.BlockSpec((1, FD), lambda i: (0, 0)),
        ],
        out_specs=pl.BlockSpec((128, FD), lambda i: (i, 0)),
        out_shape=jax.ShapeDtypeStruct((NPAD, FD), jnp.float32),
    )(acc1, y1, dis2d, W2, b1)


def _k6_body(part_ref, wm1_ref, bm1_ref, g1_ref, be1_ref,
             wm2_ref, bm2_ref, g2_ref, be2_ref, wm3_ref, bm3_ref, out_ref):
    x = jnp.max(part_ref[...], axis=(0, 1))

    def bn_relu(h, g, be):
        m = jnp.mean(h, axis=0, keepdims=True)
        v = jnp.mean((h - m) ** 2, axis=0, keepdims=True)
        return jnp.maximum(g * (h - m) / jnp.sqrt(v + 1e-5) + be, 0.0)

    h = jnp.dot(x, wm1_ref[...], preferred_element_type=jnp.float32) + bm1_ref[...]
    h = bn_relu(h, g1_ref[...], be1_ref[...])
    h = jnp.dot(h, wm2_ref[...], preferred_element_type=jnp.float32) + bm2_ref[...]
    h = bn_relu(h, g2_ref[...], be2_ref[...])
    out_ref[...] = jnp.dot(h, wm3_ref[...], preferred_element_type=jnp.float32) + bm3_ref[...]


def _k6(part, Wm1, bm1, g1, be1, Wm2, bm2, g2, be2, Wm3, bm3):
    return pl.pallas_call(
        _k6_body,
        out_shape=jax.ShapeDtypeStruct((BB, 10), jnp.float32),
    )(part, Wm1, bm1, g1, be1, Wm2, bm2, g2, be2, Wm3, bm3)


def kernel(pos, edge_index, edge_attr, batch,
           W1, b1, W2, b2, Wm1, bm1, g1, be1, Wm2, bm2, g2, be2, Wm3, bm3):
    pe = EPAD - EE
    rowp = jnp.concatenate([edge_index[0], jnp.zeros((pe,), jnp.int32)]).reshape(ER, KC)
    colp = jnp.concatenate([edge_index[1], jnp.zeros((pe,), jnp.int32)]).reshape(ER, KC)
    wp = jnp.concatenate([edge_attr, jnp.zeros((pe,), jnp.float32)]).reshape(ER, KC)
    posp = jnp.concatenate([pos, jnp.zeros((NPAD - NN, 3), jnp.float32)])
    batp = jnp.concatenate([batch, jnp.full((NPAD - NN,), BB - 1, jnp.int32)])

    deg = _deg_call(colp, wp)
    dis2d, y1 = _k2(posp, deg.reshape(NPAD, 1), W1)
    acc1 = _conv_call(y1, rowp, colp, wp)
    y2 = _k4(acc1, y1, dis2d, W2, b1.reshape(1, FD))
    part = _conv_segmax_call(y2, rowp, colp, wp, dis2d.reshape(NPAD), batp, b2)
    return _k6(part, Wm1, bm1.reshape(1, FD), g1.reshape(1, FD), be1.reshape(1, FD),
               Wm2, bm2.reshape(1, FD), g2.reshape(1, FD), be2.reshape(1, FD),
               Wm3, bm3.reshape(1, 10))


# speedup vs baseline: 8.7076x; 1.2228x over previous
import jax
import jax.numpy as jnp
from jax import lax
from jax.experimental import pallas as pl
from jax.experimental.pallas import tpu as pltpu
from jax.experimental.pallas import tpu_sc as plsc

NN = 50000
EE = 800000
BB = 64
FD = 64

NSC = 2
NTL = 16

NH = 25088
NPAD = NH * NSC
TBL = NH + 128
RT = NH // NTL
ZT = TBL // NTL

KC = 112
SUB = 8
OUTER = 56
EPT = OUTER * SUB * KC
EPAD = EPT * NTL
ER = EPAD // KC
ERT = ER // NTL

RC = 112

_mesh = plsc.VectorSubcoreMesh(core_axis_name="c", subcore_axis_name="s")
_sc_params = pltpu.CompilerParams(use_tc_tiling_on_sc=False, needs_layout_passes=False)


def _zero_rows(buf, rows):
    z = jnp.zeros((16,), jnp.float32)

    def body(k, _):
        for q in range(FD // 16):
            buf[k, pl.ds(q * 16, 16)] = z
        return _

    lax.fori_loop(0, rows, body, None)


def _zero_shared2(zbuf, sh, s):
    base = s * ZT
    nfull = ZT // KC
    rem = ZT - nfull * KC
    for i in range(nfull):
        pltpu.sync_copy(zbuf.at[pl.ds(0, KC)], sh.at[pl.ds(base + i * KC, KC)])
    pltpu.sync_copy(zbuf.at[pl.ds(0, rem)], sh.at[pl.ds(base + nfull * KC, rem)])


def _convert_cols(epk, c):
    base = c * NH
    for j in range(SUB):
        for q in range(KC // 16):
            sl = pl.ds(q * 16, 16)
            v = epk[j, 1, sl] - base
            ok = (v >= 0) & (v < NH)
            epk[j, 1, sl] = jnp.where(ok, v, NH)


def _edge_scatter_pass(y_hbm, epk_hbm, acc_sh, epk, rb0, rb1, rb2,
                       semi, sg0, sg1, sg2, ss0, ss1, ss2, c, s):
    rbufs = (rb0, rb1, rb2)
    semg = (sg0, sg1, sg2)
    sems = (ss0, ss1, ss2)

    def outer(ch, _):
        r0 = s * ERT + ch * SUB
        pltpu.async_copy(epk_hbm.at[pl.ds(r0, SUB)], epk, semi).wait()
        _convert_cols(epk, c)
        dg = [None, None, None]
        ds_ = [None, None, None]
        dg[0] = pltpu.async_copy(y_hbm.at[epk.at[0, 0]], rbufs[0], semg[0])
        for j in range(SUB):
            p = j % 3
            if j < SUB - 1:
                pn = (j + 1) % 3
                if ds_[pn] is not None:
                    ds_[pn].wait()
                dg[pn] = pltpu.async_copy(
                    y_hbm.at[epk.at[j + 1, 0]], rbufs[pn], semg[pn])
            dg[p].wait()

            def mul(g, _):
                wv = plsc.bitcast(epk[j, 2, pl.ds(g * 16, 16)], jnp.float32)
                kb = g * 16
                for u in range(16):
                    wk = wv[u]
                    for q in range(FD // 16):
                        sl = pl.ds(q * 16, 16)
                        rbufs[p][kb + u, sl] = rbufs[p][kb + u, sl] * wk
                return _

            lax.fori_loop(0, KC // 16, mul, None)
            ds_[p] = pltpu.async_copy(
                rbufs[p], acc_sh.at[epk.at[j, 1]], sems[p], add=True)
        for p in range(3):
            ds_[p].wait()
        return _

    lax.fori_loop(0, OUTER, outer, None)


def _deg_body(epk_hbm, deg_hbm, deg_sh, epk, wf, zbuf, semi, sems):
    c = lax.axis_index("c")
    s = lax.axis_index("s")
    z = jnp.zeros((16,), jnp.float32)
    for q in range(KC // 16):
        zbuf[pl.ds(q * 16, 16)] = z
    base = s * ZT
    nfull = ZT // KC
    rem = ZT - nfull * KC
    for i in range(nfull):
        pltpu.sync_copy(zbuf, deg_sh.at[pl.ds(base + i * KC, KC)])
    pltpu.sync_copy(zbuf.at[pl.ds(0, rem)], deg_sh.at[pl.ds(base + nfull * KC, rem)])
    plsc.subcore_barrier()

    def outer(ch, _):
        r0 = s * ERT + ch * SUB
        pltpu.async_copy(epk_hbm.at[pl.ds(r0, SUB)], epk, semi).wait()
        _convert_cols(epk, c)
        for j in range(SUB):
            for q in range(KC // 16):
                sl = pl.ds(q * 16, 16)
                wf[j, sl] = plsc.bitcast(epk[j, 2, sl], jnp.float32)
        ds_ = []
        for j in range(SUB):
            ds_.append(pltpu.async_copy(
                wf.at[j], deg_sh.at[epk.at[j, 1]], sems, add=True))
        for d in ds_:
            d.wait()
        return _

    lax.fori_loop(0, OUTER, outer, None)
    plsc.subcore_barrier()
    for i in range(RT // KC):
        pltpu.sync_copy(deg_sh.at[pl.ds(s * RT + i * KC, KC)], zbuf)
        pltpu.sync_copy(zbuf, deg_hbm.at[pl.ds(c * NH + s * RT + i * KC, KC)])


_deg_call = pl.kernel(
    _deg_body,
    out_type=jax.ShapeDtypeStruct((NPAD,), jnp.float32),
    mesh=_mesh,
    compiler_params=_sc_params,
    scratch_types=[
        pltpu.VMEM_SHARED((TBL,), jnp.float32),
        pltpu.VMEM((SUB, 3, KC), jnp.int32),
        pltpu.VMEM((SUB, KC), jnp.float32),
        pltpu.VMEM((KC,), jnp.float32),
        pltpu.SemaphoreType.DMA,
        pltpu.SemaphoreType.DMA,
    ],
)


def _conv_body(y_hbm, epk_hbm, acc_hbm, acc_sh, epk, rb0, rb1, rb2,
               semi, sg0, sg1, sg2, ss0, ss1, ss2):
    c = lax.axis_index("c")
    s = lax.axis_index("s")
    _zero_rows(rb0, KC)
    _zero_shared2(rb0, acc_sh, s)
    plsc.subcore_barrier()
    _edge_scatter_pass(y_hbm, epk_hbm, acc_sh, epk, rb0, rb1, rb2,
                       semi, sg0, sg1, sg2, ss0, ss1, ss2, c, s)
    plsc.subcore_barrier()
    pltpu.sync_copy(acc_sh.at[pl.ds(s * RT, RT)],
                    acc_hbm.at[pl.ds(c * NH + s * RT, RT)])


_conv_call = pl.kernel(
    _conv_body,
    out_type=jax.ShapeDtypeStruct((NPAD, FD), jnp.float32),
    mesh=_mesh,
    compiler_params=_sc_params,
    scratch_types=[
        pltpu.VMEM_SHARED((TBL, FD), jnp.float32),
        pltpu.VMEM((SUB, 3, KC), jnp.int32),
        pltpu.VMEM((KC, FD), jnp.float32),
        pltpu.VMEM((KC, FD), jnp.float32),
        pltpu.VMEM((KC, FD), jnp.float32),
        pltpu.SemaphoreType.DMA,
        pltpu.SemaphoreType.DMA,
        pltpu.SemaphoreType.DMA,
        pltpu.SemaphoreType.DMA,
        pltpu.SemaphoreType.DMA,
        pltpu.SemaphoreType.DMA,
        pltpu.SemaphoreType.DMA,
    ],
)


def _conv_segmax_body(y_hbm, epk_hbm, dis_hbm, bat_hbm, b2_hbm,
                      part_hbm, acc_sh, epk, rb0, rb1, rb2,
                      dbuf, bbuf, b2buf, segbuf,
                      semi, sg0, sg1, sg2, ss0, ss1, ss2):
    c = lax.axis_index("c")
    s = lax.axis_index("s")
    _zero_rows(rb0, KC)
    _zero_shared2(rb0, acc_sh, s)
    plsc.subcore_barrier()
    _edge_scatter_pass(y_hbm, epk_hbm, acc_sh, epk, rb0, rb1, rb2,
                       semi, sg0, sg1, sg2, ss0, ss1, ss2, c, s)
    plsc.subcore_barrier()

    ninf = jnp.full((16,), -jnp.inf, jnp.float32)

    def seg_init(k, _):
        for q in range(FD // 16):
            segbuf[k, pl.ds(q * 16, 16)] = ninf
        return _

    lax.fori_loop(0, BB, seg_init, None)

    pltpu.sync_copy(b2_hbm, b2buf)
    gstart = c * NH + s * RT

    def seg_chunk(i, _):
        l0 = s * RT + i * RC
        g0 = gstart + i * RC
        pltpu.sync_copy(acc_sh.at[pl.ds(l0, RC)], rb0.at[pl.ds(0, RC)])
        pltpu.sync_copy(y_hbm.at[pl.ds(g0, RC)], rb1.at[pl.ds(0, RC)])
        pltpu.sync_copy(dis_hbm.at[pl.ds(g0, RC)], dbuf)
        pltpu.sync_copy(bat_hbm.at[pl.ds(g0, RC)], bbuf)

        def seg_grp(g, _):
            dv = dbuf[pl.ds(g * 16, 16)]
            bv = bbuf[pl.ds(g * 16, 16)]
            kb = g * 16
            gbase = g0 + kb
            for u in range(16):
                dk = dv[u]
                bid = bv[u]
                pen = jnp.where(gbase + u < NN, 0.0, -jnp.inf).astype(jnp.float32)
                for q in range(FD // 16):
                    sl = pl.ds(q * 16, 16)
                    v = dk * (rb0[kb + u, sl] + rb1[kb + u, sl]) + b2buf[sl] + pen
                    segbuf[bid, sl] = jnp.maximum(segbuf[bid, sl], v)
            return _

        lax.fori_loop(0, RC // 16, seg_grp, None)
        return _

    lax.fori_loop(0, RT // RC, seg_chunk, None)
    pltpu.sync_copy(segbuf, part_hbm.at[c, s])


_conv_segmax_call = pl.kernel(
    _conv_segmax_body,
    out_type=jax.ShapeDtypeStruct((NSC, NTL, BB, FD), jnp.float32),
    mesh=_mesh,
    compiler_params=_sc_params,
    scratch_types=[
        pltpu.VMEM_SHARED((TBL, FD), jnp.float32),
        pltpu.VMEM((SUB, 3, KC), jnp.int32),
        pltpu.VMEM((KC, FD), jnp.float32),
        pltpu.VMEM((KC, FD), jnp.float32),
        pltpu.VMEM((KC, FD), jnp.float32),
        pltpu.VMEM((RC,), jnp.float32),
        pltpu.VMEM((RC,), jnp.int32),
        pltpu.VMEM((FD,), jnp.float32),
        pltpu.VMEM((BB, FD), jnp.float32),
        pltpu.SemaphoreType.DMA,
        pltpu.SemaphoreType.DMA,
        pltpu.SemaphoreType.DMA,
        pltpu.SemaphoreType.DMA,
        pltpu.SemaphoreType.DMA,
        pltpu.SemaphoreType.DMA,
        pltpu.SemaphoreType.DMA,
    ],
)


_NB = NPAD // 128


def _k2_body(pos_ref, deg_ref, w1_ref, dis_ref, y1_ref):
    dis = lax.rsqrt(deg_ref[...] + 1.0)
    dis_ref[...] = dis
    xl = jnp.dot(pos_ref[...], w1_ref[...], preferred_element_type=jnp.float32)
    y1_ref[...] = xl * dis


def _k2(posp, deg2d, W1):
    return pl.pallas_call(
        _k2_body,
        grid=(_NB,),
        in_specs=[
            pl.BlockSpec((128, 3), lambda i: (i, 0)),
            pl.BlockSpec((128, 1), lambda i: (i, 0)),
            pl.BlockSpec((3, FD), lambda i: (0, 0)),
        ],
        out_specs=[
            pl.BlockSpec((128, 1), lambda i: (i, 0)),
            pl.BlockSpec((128, FD), lambda i: (i, 0)),
        ],
        out_shape=[
            jax.ShapeDtypeStruct((NPAD, 1), jnp.float32),
            jax.ShapeDtypeStruct((NPAD, FD), jnp.float32),
        ],
    )(posp, deg2d, W1)


def _k4_body(acc_ref, y1_ref, dis_ref, w2_ref, b1_ref, y2_ref):
    dis = dis_ref[...]
    x1 = jnp.maximum(dis * (acc_ref[...] + y1_ref[...]) + b1_ref[...], 0.0)
    y2_ref[...] = jnp.dot(x1, w2_ref[...], preferred_element_type=jnp.float32) * dis


def _k4(acc1, y1, dis2d, W2, b1):
    return pl.pallas_call(
        _k4_body,
        grid=(_NB,),
        in_specs=[
            pl.BlockSpec((128, FD), lambda i: (i, 0)),
            pl.BlockSpec((128, FD), lambda i: (i, 0)),
            pl.BlockSpec((128, 1), lambda i: (i, 0)),
            pl.BlockSpec((FD, FD), lambda i: (0, 0)),
            pl.BlockSpec((1, FD), lambda i: (0, 0)),
        ],
        out_specs=pl.BlockSpec((128, FD), lambda i: (i, 0)),
        out_shape=jax.ShapeDtypeStruct((NPAD, FD), jnp.float32),
    )(acc1, y1, dis2d, W2, b1)


def _k6_body(part_ref, wm1_ref, bm1_ref, g1_ref, be1_ref,
             wm2_ref, bm2_ref, g2_ref, be2_ref, wm3_ref, bm3_ref, out_ref):
    x = jnp.max(part_ref[...], axis=(0, 1))

    def bn_relu(h, g, be):
        m = jnp.mean(h, axis=0, keepdims=True)
        v = jnp.mean((h - m) ** 2, axis=0, keepdims=True)
        return jnp.maximum(g * (h - m) / jnp.sqrt(v + 1e-5) + be, 0.0)

    h = jnp.dot(x, wm1_ref[...], preferred_element_type=jnp.float32) + bm1_ref[...]
    h = bn_relu(h, g1_ref[...], be1_ref[...])
    h = jnp.dot(h, wm2_ref[...], preferred_element_type=jnp.float32) + bm2_ref[...]
    h = bn_relu(h, g2_ref[...], be2_ref[...])
    out_ref[...] = jnp.dot(h, wm3_ref[...], preferred_element_type=jnp.float32) + bm3_ref[...]


def _k6(part, Wm1, bm1, g1, be1, Wm2, bm2, g2, be2, Wm3, bm3):
    return pl.pallas_call(
        _k6_body,
        out_shape=jax.ShapeDtypeStruct((BB, 10), jnp.float32),
    )(part, Wm1, bm1, g1, be1, Wm2, bm2, g2, be2, Wm3, bm3)


def kernel(pos, edge_index, edge_attr, batch,
           W1, b1, W2, b2, Wm1, bm1, g1, be1, Wm2, bm2, g2, be2, Wm3, bm3):
    pe = EPAD - EE
    zpad = jnp.zeros((pe,), jnp.int32)
    rowp = jnp.concatenate([edge_index[0], zpad]).reshape(ER, KC)
    colp = jnp.concatenate([edge_index[1], zpad]).reshape(ER, KC)
    wp = jnp.concatenate([lax.bitcast_convert_type(edge_attr, jnp.int32),
                          zpad]).reshape(ER, KC)
    epack = jnp.stack([rowp, colp, wp], axis=1)
    posp = jnp.concatenate([pos, jnp.zeros((NPAD - NN, 3), jnp.float32)])
    batp = jnp.concatenate([batch, jnp.full((NPAD - NN,), BB - 1, jnp.int32)])

    deg = _deg_call(epack)
    dis2d, y1 = _k2(posp, deg.reshape(NPAD, 1), W1)
    acc1 = _conv_call(y1, epack)
    y2 = _k4(acc1, y1, dis2d, W2, b1.reshape(1, FD))
    part = _conv_segmax_call(y2, epack, dis2d.reshape(NPAD), batp, b2)
    return _k6(part, Wm1, bm1.reshape(1, FD), g1.reshape(1, FD), be1.reshape(1, FD),
               Wm2, bm2.reshape(1, FD), g2.reshape(1, FD), be2.reshape(1, FD),
               Wm3, bm3.reshape(1, 10))


# speedup vs baseline: 11.7361x; 1.3478x over previous
import jax
import jax.numpy as jnp
from jax import lax
from jax.experimental import pallas as pl
from jax.experimental.pallas import tpu as pltpu
from jax.experimental.pallas import tpu_sc as plsc

NN = 50000
EE = 800000
BB = 64
FD = 64

NSC = 2
NTL = 16

NH = 25088
NPAD = NH * NSC
TBL = NH + 128
RT = NH // NTL
ZT = TBL // NTL

KC = 112
SUB = 8
OUTER = 56
EPT = OUTER * SUB * KC
EPAD = EPT * NTL
ER = EPAD // KC
ERT = ER // NTL

RC = 112

_mesh = plsc.VectorSubcoreMesh(core_axis_name="c", subcore_axis_name="s")
_sc_params = pltpu.CompilerParams(use_tc_tiling_on_sc=False, needs_layout_passes=False)


def _zero_rows(buf, rows):
    z = jnp.zeros((16,), jnp.float32)

    def body(k, _):
        for q in range(FD // 16):
            buf[k, pl.ds(q * 16, 16)] = z
        return _

    lax.fori_loop(0, rows, body, None)


def _zero_shared2(zbuf, sh, s):
    base = s * ZT
    nfull = ZT // KC
    rem = ZT - nfull * KC
    for i in range(nfull):
        pltpu.sync_copy(zbuf.at[pl.ds(0, KC)], sh.at[pl.ds(base + i * KC, KC)])
    pltpu.sync_copy(zbuf.at[pl.ds(0, rem)], sh.at[pl.ds(base + nfull * KC, rem)])


def _convert_cols(epk, c):
    base = c * NH
    for j in range(SUB):
        for q in range(KC // 16):
            sl = pl.ds(q * 16, 16)
            v = epk[j, 1, sl] - base
            ok = (v >= 0) & (v < NH)
            epk[j, 1, sl] = jnp.where(ok, v, NH)


def _edge_scatter_pass(y_hbm, epk_hbm, acc_sh, epk, rb0, rb1, rb2,
                       semi, sg0, sg1, sg2, ss0, ss1, ss2, c, s):
    rbufs = (rb0, rb1, rb2)
    semg = (sg0, sg1, sg2)
    sems = (ss0, ss1, ss2)

    def outer(ch, _):
        r0 = s * ERT + ch * SUB
        pltpu.async_copy(epk_hbm.at[pl.ds(r0, SUB)], epk, semi).wait()
        _convert_cols(epk, c)
        dg = [None, None, None]
        ds_ = [None, None, None]
        dg[0] = pltpu.async_copy(y_hbm.at[epk.at[0, 0]], rbufs[0], semg[0])
        for j in range(SUB):
            p = j % 3
            if j < SUB - 1:
                pn = (j + 1) % 3
                if ds_[pn] is not None:
                    ds_[pn].wait()
                dg[pn] = pltpu.async_copy(
                    y_hbm.at[epk.at[j + 1, 0]], rbufs[pn], semg[pn])
            dg[p].wait()

            def mul(g, _):
                wv = plsc.bitcast(epk[j, 2, pl.ds(g * 16, 16)], jnp.float32)
                kb = g * 16
                for u in range(16):
                    wk = wv[u]
                    for q in range(FD // 16):
                        sl = pl.ds(q * 16, 16)
                        rbufs[p][kb + u, sl] = rbufs[p][kb + u, sl] * wk
                return _

            lax.fori_loop(0, KC // 16, mul, None)
            ds_[p] = pltpu.async_copy(
                rbufs[p], acc_sh.at[epk.at[j, 1]], sems[p], add=True)
        for p in range(3):
            ds_[p].wait()
        return _

    lax.fori_loop(0, OUTER, outer, None)


DPAD = NPAD + 128
DZT = DPAD // NTL
DWT = NPAD // NTL
DOUT = OUTER // 2


def _deg_body(epk_hbm, deg_hbm, deg_sh, epk, wf, zbuf, semi, sems):
    c = lax.axis_index("c")
    s = lax.axis_index("s")
    z = jnp.zeros((16,), jnp.float32)
    for q in range(KC // 16):
        zbuf[pl.ds(q * 16, 16)] = z
    base = s * DZT
    nfull = DZT // KC
    rem = DZT - nfull * KC
    for i in range(nfull):
        pltpu.sync_copy(zbuf, deg_sh.at[pl.ds(base + i * KC, KC)])
    pltpu.sync_copy(zbuf.at[pl.ds(0, rem)], deg_sh.at[pl.ds(base + nfull * KC, rem)])
    plsc.subcore_barrier()

    tbase = c * (ER // 2) + s * (ERT // 2)

    def prep(ch, b):
        pltpu.async_copy(epk_hbm.at[pl.ds(tbase + ch * SUB, SUB)],
                         epk.at[b], semi).wait()
        for j in range(SUB):
            for q in range(KC // 16):
                sl = pl.ds(q * 16, 16)
                wf[b, j, sl] = plsc.bitcast(epk[b, j, 2, sl], jnp.float32)

    def fire(b):
        return [pltpu.async_copy(wf.at[b, j], deg_sh.at[epk.at[b, j, 1]],
                                 sems, add=True) for j in range(SUB)]

    def drain(ds_):
        for d in ds_:
            d.wait()

    def outer(i, _):
        ch = i * 4
        prep(ch, 0)
        dsA = fire(0)
        prep(ch + 1, 1)
        drain(dsA)
        dsB = fire(1)
        prep(ch + 2, 0)
        drain(dsB)
        dsC = fire(0)
        prep(ch + 3, 1)
        drain(dsC)
        dsD = fire(1)
        drain(dsD)
        return _

    lax.fori_loop(0, DOUT // 4, outer, None)
    plsc.subcore_barrier()
    for i in range(DWT // KC):
        pltpu.sync_copy(deg_sh.at[pl.ds(s * DWT + i * KC, KC)], zbuf)
        pltpu.sync_copy(zbuf, deg_hbm.at[c, pl.ds(s * DWT + i * KC, KC)])


_deg_call = pl.kernel(
    _deg_body,
    out_type=jax.ShapeDtypeStruct((NSC, NPAD), jnp.float32),
    mesh=_mesh,
    compiler_params=_sc_params,
    scratch_types=[
        pltpu.VMEM_SHARED((DPAD,), jnp.float32),
        pltpu.VMEM((2, SUB, 3, KC), jnp.int32),
        pltpu.VMEM((2, SUB, KC), jnp.float32),
        pltpu.VMEM((KC,), jnp.float32),
        pltpu.SemaphoreType.DMA,
        pltpu.SemaphoreType.DMA,
    ],
)


def _conv_body(y_hbm, epk_hbm, acc_hbm, acc_sh, epk, rb0, rb1, rb2,
               semi, sg0, sg1, sg2, ss0, ss1, ss2):
    c = lax.axis_index("c")
    s = lax.axis_index("s")
    _zero_rows(rb0, KC)
    _zero_shared2(rb0, acc_sh, s)
    plsc.subcore_barrier()
    _edge_scatter_pass(y_hbm, epk_hbm, acc_sh, epk, rb0, rb1, rb2,
                       semi, sg0, sg1, sg2, ss0, ss1, ss2, c, s)
    plsc.subcore_barrier()
    pltpu.sync_copy(acc_sh.at[pl.ds(s * RT, RT)],
                    acc_hbm.at[pl.ds(c * NH + s * RT, RT)])


_conv_call = pl.kernel(
    _conv_body,
    out_type=jax.ShapeDtypeStruct((NPAD, FD), jnp.float32),
    mesh=_mesh,
    compiler_params=_sc_params,
    scratch_types=[
        pltpu.VMEM_SHARED((TBL, FD), jnp.float32),
        pltpu.VMEM((SUB, 3, KC), jnp.int32),
        pltpu.VMEM((KC, FD), jnp.float32),
        pltpu.VMEM((KC, FD), jnp.float32),
        pltpu.VMEM((KC, FD), jnp.float32),
        pltpu.SemaphoreType.DMA,
        pltpu.SemaphoreType.DMA,
        pltpu.SemaphoreType.DMA,
        pltpu.SemaphoreType.DMA,
        pltpu.SemaphoreType.DMA,
        pltpu.SemaphoreType.DMA,
        pltpu.SemaphoreType.DMA,
    ],
)


def _conv_segmax_body(y_hbm, epk_hbm, dis_hbm, bat_hbm, b2_hbm,
                      part_hbm, acc_sh, epk, rb0, rb1, rb2,
                      dbuf, bbuf, b2buf, segbuf,
                      semi, sg0, sg1, sg2, ss0, ss1, ss2):
    c = lax.axis_index("c")
    s = lax.axis_index("s")
    _zero_rows(rb0, KC)
    _zero_shared2(rb0, acc_sh, s)
    plsc.subcore_barrier()
    _edge_scatter_pass(y_hbm, epk_hbm, acc_sh, epk, rb0, rb1, rb2,
                       semi, sg0, sg1, sg2, ss0, ss1, ss2, c, s)
    plsc.subcore_barrier()

    ninf = jnp.full((16,), -jnp.inf, jnp.float32)

    def seg_init(k, _):
        for q in range(FD // 16):
            segbuf[k, pl.ds(q * 16, 16)] = ninf
        return _

    lax.fori_loop(0, BB, seg_init, None)

    pltpu.sync_copy(b2_hbm, b2buf)
    gstart = c * NH + s * RT

    def seg_chunk(i, _):
        l0 = s * RT + i * RC
        g0 = gstart + i * RC
        pltpu.sync_copy(acc_sh.at[pl.ds(l0, RC)], rb0.at[pl.ds(0, RC)])
        pltpu.sync_copy(y_hbm.at[pl.ds(g0, RC)], rb1.at[pl.ds(0, RC)])
        pltpu.sync_copy(dis_hbm.at[pl.ds(g0, RC)], dbuf)
        pltpu.sync_copy(bat_hbm.at[pl.ds(g0, RC)], bbuf)

        def seg_grp(g, _):
            dv = dbuf[pl.ds(g * 16, 16)]
            bv = bbuf[pl.ds(g * 16, 16)]
            kb = g * 16
            gbase = g0 + kb
            for u in range(16):
                dk = dv[u]
                bid = bv[u]
                pen = jnp.where(gbase + u < NN, 0.0, -jnp.inf).astype(jnp.float32)
                for q in range(FD // 16):
                    sl = pl.ds(q * 16, 16)
                    v = dk * (rb0[kb + u, sl] + rb1[kb + u, sl]) + b2buf[sl] + pen
                    segbuf[bid, sl] = jnp.maximum(segbuf[bid, sl], v)
            return _

        lax.fori_loop(0, RC // 16, seg_grp, None)
        return _

    lax.fori_loop(0, RT // RC, seg_chunk, None)
    pltpu.sync_copy(segbuf, part_hbm.at[c, s])


_conv_segmax_call = pl.kernel(
    _conv_segmax_body,
    out_type=jax.ShapeDtypeStruct((NSC, NTL, BB, FD), jnp.float32),
    mesh=_mesh,
    compiler_params=_sc_params,
    scratch_types=[
        pltpu.VMEM_SHARED((TBL, FD), jnp.float32),
        pltpu.VMEM((SUB, 3, KC), jnp.int32),
        pltpu.VMEM((KC, FD), jnp.float32),
        pltpu.VMEM((KC, FD), jnp.float32),
        pltpu.VMEM((KC, FD), jnp.float32),
        pltpu.VMEM((RC,), jnp.float32),
        pltpu.VMEM((RC,), jnp.int32),
        pltpu.VMEM((FD,), jnp.float32),
        pltpu.VMEM((BB, FD), jnp.float32),
        pltpu.SemaphoreType.DMA,
        pltpu.SemaphoreType.DMA,
        pltpu.SemaphoreType.DMA,
        pltpu.SemaphoreType.DMA,
        pltpu.SemaphoreType.DMA,
        pltpu.SemaphoreType.DMA,
        pltpu.SemaphoreType.DMA,
    ],
)


_RB = 1024
_NG = NPAD // _RB


def _k2_body(pos_ref, deg_ref, w1_ref, dis_ref, y1_ref):
    deg = deg_ref[0] + deg_ref[1]
    dis = lax.rsqrt(deg + 1.0)
    dis_ref[...] = dis
    xl = jnp.dot(pos_ref[...], w1_ref[...], preferred_element_type=jnp.float32)
    y1_ref[...] = xl * dis


def _k2(posp, deg3d, W1):
    return pl.pallas_call(
        _k2_body,
        grid=(_NG,),
        in_specs=[
            pl.BlockSpec((_RB, 3), lambda i: (i, 0)),
            pl.BlockSpec((NSC, _RB, 1), lambda i: (0, i, 0)),
            pl.BlockSpec((3, FD), lambda i: (0, 0)),
        ],
        out_specs=[
            pl.BlockSpec((_RB, 1), lambda i: (i, 0)),
            pl.BlockSpec((_RB, FD), lambda i: (i, 0)),
        ],
        out_shape=[
            jax.ShapeDtypeStruct((NPAD, 1), jnp.float32),
            jax.ShapeDtypeStruct((NPAD, FD), jnp.float32),
        ],
    )(posp, deg3d, W1)


def _k4_body(acc_ref, y1_ref, dis_ref, w2_ref, b1_ref, y2_ref):
    dis = dis_ref[...]
    x1 = jnp.maximum(dis * (acc_ref[...] + y1_ref[...]) + b1_ref[...], 0.0)
    y2_ref[...] = jnp.dot(x1, w2_ref[...], preferred_element_type=jnp.float32) * dis


def _k4(acc1, y1, dis2d, W2, b1):
    return pl.pallas_call(
        _k4_body,
        grid=(_NG,),
        in_specs=[
            pl.BlockSpec((_RB, FD), lambda i: (i, 0)),
            pl.BlockSpec((_RB, FD), lambda i: (i, 0)),
            pl.BlockSpec((_RB, 1), lambda i: (i, 0)),
            pl.BlockSpec((FD, FD), lambda i: (0, 0)),
            pl.BlockSpec((1, FD), lambda i: (0, 0)),
        ],
        out_specs=pl.BlockSpec((_RB, FD), lambda i: (i, 0)),
        out_shape=jax.ShapeDtypeStruct((NPAD, FD), jnp.float32),
    )(acc1, y1, dis2d, W2, b1)


def _k6_body(part_ref, wm1_ref, bm1_ref, g1_ref, be1_ref,
             wm2_ref, bm2_ref, g2_ref, be2_ref, wm3_ref, bm3_ref, out_ref):
    x = jnp.max(part_ref[...], axis=(0, 1))

    def bn_relu(h, g, be):
        m = jnp.mean(h, axis=0, keepdims=True)
        v = jnp.mean((h - m) ** 2, axis=0, keepdims=True)
        return jnp.maximum(g * (h - m) / jnp.sqrt(v + 1e-5) + be, 0.0)

    h = jnp.dot(x, wm1_ref[...], preferred_element_type=jnp.float32) + bm1_ref[...]
    h = bn_relu(h, g1_ref[...], be1_ref[...])
    h = jnp.dot(h, wm2_ref[...], preferred_element_type=jnp.float32) + bm2_ref[...]
    h = bn_relu(h, g2_ref[...], be2_ref[...])
    out_ref[...] = jnp.dot(h, wm3_ref[...], preferred_element_type=jnp.float32) + bm3_ref[...]


def _k6(part, Wm1, bm1, g1, be1, Wm2, bm2, g2, be2, Wm3, bm3):
    return pl.pallas_call(
        _k6_body,
        out_shape=jax.ShapeDtypeStruct((BB, 10), jnp.float32),
    )(part, Wm1, bm1, g1, be1, Wm2, bm2, g2, be2, Wm3, bm3)


def kernel(pos, edge_index, edge_attr, batch,
           W1, b1, W2, b2, Wm1, bm1, g1, be1, Wm2, bm2, g2, be2, Wm3, bm3):
    pe = EPAD - EE
    zpad = jnp.zeros((pe,), jnp.int32)
    rowp = jnp.concatenate([edge_index[0], zpad]).reshape(ER, KC)
    colp = jnp.concatenate([edge_index[1], zpad]).reshape(ER, KC)
    wp = jnp.concatenate([lax.bitcast_convert_type(edge_attr, jnp.int32),
                          zpad]).reshape(ER, KC)
    epack = jnp.stack([rowp, colp, wp], axis=1)
    posp = jnp.concatenate([pos, jnp.zeros((NPAD - NN, 3), jnp.float32)])
    batp = jnp.concatenate([batch, jnp.full((NPAD - NN,), BB - 1, jnp.int32)])

    deg = _deg_call(epack)
    dis2d, y1 = _k2(posp, deg.reshape(NSC, NPAD, 1), W1)
    acc1 = _conv_call(y1, epack)
    y2 = _k4(acc1, y1, dis2d, W2, b1.reshape(1, FD))
    part = _conv_segmax_call(y2, epack, dis2d.reshape(NPAD), batp, b2)
    return _k6(part, Wm1, bm1.reshape(1, FD), g1.reshape(1, FD), be1.reshape(1, FD),
               Wm2, bm2.reshape(1, FD), g2.reshape(1, FD), be2.reshape(1, FD),
               Wm3, bm3.reshape(1, 10))


# speedup vs baseline: 20.9485x; 1.7850x over previous
import jax
import jax.numpy as jnp
from jax import lax
from jax.experimental import pallas as pl
from jax.experimental.pallas import tpu as pltpu
from jax.experimental.pallas import tpu_sc as plsc

NN = 50000
EE = 800000
BB = 64
FD = 64

NSC = 2
NTL = 16

NPAD = 50176
FH = FD // NSC
RT = NPAD // NTL

KC = 112
SUB = 8
OUTER = 56
EPT = OUTER * SUB * KC
EPAD = EPT * NTL
ER = EPAD // KC
ERT = ER // NTL

RC = 112

_mesh = plsc.VectorSubcoreMesh(core_axis_name="c", subcore_axis_name="s")
_sc_params = pltpu.CompilerParams(use_tc_tiling_on_sc=False, needs_layout_passes=False)


def _zero_rows(buf, rows):
    z = jnp.zeros((16,), jnp.float32)

    def body(k, _):
        for q in range(FH // 16):
            buf[k, pl.ds(q * 16, 16)] = z
        return _

    lax.fori_loop(0, rows, body, None)


def _zero_shared2(zbuf, sh, s):
    base = s * RT
    for i in range(RT // KC):
        pltpu.sync_copy(zbuf.at[pl.ds(0, KC)], sh.at[pl.ds(base + i * KC, KC)])


def _convert_rows(epk, c):
    off = c * NPAD
    for j in range(SUB):
        for q in range(KC // 16):
            sl = pl.ds(q * 16, 16)
            epk[j, 0, sl] = epk[j, 0, sl] + off


def _edge_scatter_pass(y_hbm, epk_hbm, acc_sh, epk, rb0, rb1, rb2,
                       semi, sg0, sg1, sg2, ss0, ss1, ss2, c, s):
    rbufs = (rb0, rb1, rb2)
    semg = (sg0, sg1, sg2)
    sems = (ss0, ss1, ss2)

    def outer(ch, _):
        r0 = s * ERT + ch * SUB
        pltpu.async_copy(epk_hbm.at[pl.ds(r0, SUB)], epk, semi).wait()
        _convert_rows(epk, c)
        dg = [None, None, None]
        ds_ = [None, None, None]
        dg[0] = pltpu.async_copy(y_hbm.at[epk.at[0, 0]], rbufs[0], semg[0])
        for j in range(SUB):
            p = j % 3
            if j < SUB - 1:
                pn = (j + 1) % 3
                if ds_[pn] is not None:
                    ds_[pn].wait()
                dg[pn] = pltpu.async_copy(
                    y_hbm.at[epk.at[j + 1, 0]], rbufs[pn], semg[pn])
            dg[p].wait()

            def mul(g, _):
                wv = plsc.bitcast(epk[j, 2, pl.ds(g * 16, 16)], jnp.float32)
                kb = g * 16
                for u in range(16):
                    wk = wv[u]
                    for q in range(FH // 16):
                        sl = pl.ds(q * 16, 16)
                        rbufs[p][kb + u, sl] = rbufs[p][kb + u, sl] * wk
                return _

            lax.fori_loop(0, KC // 16, mul, None)
            ds_[p] = pltpu.async_copy(
                rbufs[p], acc_sh.at[epk.at[j, 1]], sems[p], add=True)
        for p in range(3):
            ds_[p].wait()
        return _

    lax.fori_loop(0, OUTER, outer, None)


DPAD = NPAD + 128
DZT = DPAD // NTL
DWT = NPAD // NTL
DOUT = OUTER // 2


def _deg_body(epk_hbm, deg_hbm, deg_sh, epk, wf, zbuf, semi, sems):
    c = lax.axis_index("c")
    s = lax.axis_index("s")
    z = jnp.zeros((16,), jnp.float32)
    for q in range(KC // 16):
        zbuf[pl.ds(q * 16, 16)] = z
    base = s * DZT
    nfull = DZT // KC
    rem = DZT - nfull * KC
    for i in range(nfull):
        pltpu.sync_copy(zbuf, deg_sh.at[pl.ds(base + i * KC, KC)])
    pltpu.sync_copy(zbuf.at[pl.ds(0, rem)], deg_sh.at[pl.ds(base + nfull * KC, rem)])
    plsc.subcore_barrier()

    tbase = c * (ER // 2) + s * (ERT // 2)

    def prep(ch, b):
        pltpu.async_copy(epk_hbm.at[pl.ds(tbase + ch * SUB, SUB)],
                         epk.at[b], semi).wait()
        for j in range(SUB):
            for q in range(KC // 16):
                sl = pl.ds(q * 16, 16)
                wf[b, j, sl] = plsc.bitcast(epk[b, j, 2, sl], jnp.float32)

    def fire(b):
        return [pltpu.async_copy(wf.at[b, j], deg_sh.at[epk.at[b, j, 1]],
                                 sems, add=True) for j in range(SUB)]

    def drain(ds_):
        for d in ds_:
            d.wait()

    def outer(i, _):
        ch = i * 4
        prep(ch, 0)
        dsA = fire(0)
        prep(ch + 1, 1)
        drain(dsA)
        dsB = fire(1)
        prep(ch + 2, 0)
        drain(dsB)
        dsC = fire(0)
        prep(ch + 3, 1)
        drain(dsC)
        dsD = fire(1)
        drain(dsD)
        return _

    lax.fori_loop(0, DOUT // 4, outer, None)
    plsc.subcore_barrier()
    for i in range(DWT // KC):
        pltpu.sync_copy(deg_sh.at[pl.ds(s * DWT + i * KC, KC)], zbuf)
        pltpu.sync_copy(zbuf, deg_hbm.at[c, pl.ds(s * DWT + i * KC, KC)])


_deg_call = pl.kernel(
    _deg_body,
    out_type=jax.ShapeDtypeStruct((NSC, NPAD), jnp.float32),
    mesh=_mesh,
    compiler_params=_sc_params,
    scratch_types=[
        pltpu.VMEM_SHARED((DPAD,), jnp.float32),
        pltpu.VMEM((2, SUB, 3, KC), jnp.int32),
        pltpu.VMEM((2, SUB, KC), jnp.float32),
        pltpu.VMEM((KC,), jnp.float32),
        pltpu.SemaphoreType.DMA,
        pltpu.SemaphoreType.DMA,
    ],
)


def _conv_body(y_hbm, epk_hbm, acc_hbm, acc_sh, epk, rb0, rb1, rb2,
               semi, sg0, sg1, sg2, ss0, ss1, ss2):
    c = lax.axis_index("c")
    s = lax.axis_index("s")
    _zero_rows(rb0, KC)
    _zero_shared2(rb0, acc_sh, s)
    plsc.subcore_barrier()
    _edge_scatter_pass(y_hbm, epk_hbm, acc_sh, epk, rb0, rb1, rb2,
                       semi, sg0, sg1, sg2, ss0, ss1, ss2, c, s)
    plsc.subcore_barrier()
    pltpu.sync_copy(acc_sh.at[pl.ds(s * RT, RT)],
                    acc_hbm.at[c, pl.ds(s * RT, RT)])


_conv_call = pl.kernel(
    _conv_body,
    out_type=jax.ShapeDtypeStruct((NSC, NPAD, FH), jnp.float32),
    mesh=_mesh,
    compiler_params=_sc_params,
    scratch_types=[
        pltpu.VMEM_SHARED((NPAD, FH), jnp.float32),
        pltpu.VMEM((SUB, 3, KC), jnp.int32),
        pltpu.VMEM((KC, FH), jnp.float32),
        pltpu.VMEM((KC, FH), jnp.float32),
        pltpu.VMEM((KC, FH), jnp.float32),
        pltpu.SemaphoreType.DMA,
        pltpu.SemaphoreType.DMA,
        pltpu.SemaphoreType.DMA,
        pltpu.SemaphoreType.DMA,
        pltpu.SemaphoreType.DMA,
        pltpu.SemaphoreType.DMA,
        pltpu.SemaphoreType.DMA,
    ],
)


def _conv_segmax_body(y_hbm, epk_hbm, dis_hbm, bat_hbm, b2_hbm,
                      part_hbm, acc_sh, epk, rb0, rb1, rb2,
                      dbuf, bbuf, b2buf, segbuf,
                      semi, sg0, sg1, sg2, ss0, ss1, ss2):
    c = lax.axis_index("c")
    s = lax.axis_index("s")
    _zero_rows(rb0, KC)
    _zero_shared2(rb0, acc_sh, s)
    plsc.subcore_barrier()
    _edge_scatter_pass(y_hbm, epk_hbm, acc_sh, epk, rb0, rb1, rb2,
                       semi, sg0, sg1, sg2, ss0, ss1, ss2, c, s)
    plsc.subcore_barrier()

    ninf = jnp.full((16,), -jnp.inf, jnp.float32)

    def seg_init(k, _):
        for q in range(FH // 16):
            segbuf[k, pl.ds(q * 16, 16)] = ninf
        return _

    lax.fori_loop(0, BB, seg_init, None)

    pltpu.sync_copy(b2_hbm, b2buf)
    gstart = s * RT

    def seg_chunk(i, _):
        g0 = gstart + i * RC
        pltpu.sync_copy(acc_sh.at[pl.ds(g0, RC)], rb0.at[pl.ds(0, RC)])
        pltpu.sync_copy(y_hbm.at[pl.ds(c * NPAD + g0, RC)], rb1.at[pl.ds(0, RC)])
        pltpu.sync_copy(dis_hbm.at[pl.ds(g0, RC)], dbuf)
        pltpu.sync_copy(bat_hbm.at[pl.ds(g0, RC)], bbuf)

        def seg_grp(g, _):
            dv = dbuf[pl.ds(g * 16, 16)]
            bv = bbuf[pl.ds(g * 16, 16)]
            kb = g * 16
            gbase = g0 + kb
            for u in range(16):
                dk = dv[u]
                bid = bv[u]
                pen = jnp.where(gbase + u < NN, 0.0, -jnp.inf).astype(jnp.float32)
                for q in range(FH // 16):
                    sl = pl.ds(q * 16, 16)
                    b2v = b2buf[pl.ds(c * FH + q * 16, 16)]
                    v = dk * (rb0[kb + u, sl] + rb1[kb + u, sl]) + b2v + pen
                    segbuf[bid, sl] = jnp.maximum(segbuf[bid, sl], v)
            return _

        lax.fori_loop(0, RC // 16, seg_grp, None)
        return _

    lax.fori_loop(0, RT // RC, seg_chunk, None)
    pltpu.sync_copy(segbuf, part_hbm.at[c, s])


_conv_segmax_call = pl.kernel(
    _conv_segmax_body,
    out_type=jax.ShapeDtypeStruct((NSC, NTL, BB, FH), jnp.float32),
    mesh=_mesh,
    compiler_params=_sc_params,
    scratch_types=[
        pltpu.VMEM_SHARED((NPAD, FH), jnp.float32),
        pltpu.VMEM((SUB, 3, KC), jnp.int32),
        pltpu.VMEM((KC, FH), jnp.float32),
        pltpu.VMEM((KC, FH), jnp.float32),
        pltpu.VMEM((KC, FH), jnp.float32),
        pltpu.VMEM((RC,), jnp.float32),
        pltpu.VMEM((RC,), jnp.int32),
        pltpu.VMEM((FD,), jnp.float32),
        pltpu.VMEM((BB, FH), jnp.float32),
        pltpu.SemaphoreType.DMA,
        pltpu.SemaphoreType.DMA,
        pltpu.SemaphoreType.DMA,
        pltpu.SemaphoreType.DMA,
        pltpu.SemaphoreType.DMA,
        pltpu.SemaphoreType.DMA,
        pltpu.SemaphoreType.DMA,
    ],
)


_RB = 1024
_NG = NPAD // _RB


def _k2_body(pos_ref, deg_ref, w1_ref, dis_ref, y1_ref):
    deg = deg_ref[0] + deg_ref[1]
    dis = lax.rsqrt(deg + 1.0)
    dis_ref[...] = dis
    xl = jnp.dot(pos_ref[...], w1_ref[...], preferred_element_type=jnp.float32)
    y = xl * dis
    y1_ref[0] = y[:, :FH]
    y1_ref[1] = y[:, FH:]


def _k2(posp, deg3d, W1):
    return pl.pallas_call(
        _k2_body,
        grid=(_NG,),
        in_specs=[
            pl.BlockSpec((_RB, 3), lambda i: (i, 0)),
            pl.BlockSpec((NSC, _RB, 1), lambda i: (0, i, 0)),
            pl.BlockSpec((3, FD), lambda i: (0, 0)),
        ],
        out_specs=[
            pl.BlockSpec((_RB, 1), lambda i: (i, 0)),
            pl.BlockSpec((NSC, _RB, FH), lambda i: (0, i, 0)),
        ],
        out_shape=[
            jax.ShapeDtypeStruct((NPAD, 1), jnp.float32),
            jax.ShapeDtypeStruct((NSC, NPAD, FH), jnp.float32),
        ],
    )(posp, deg3d, W1)


def _k4_body(acc_ref, y1_ref, dis_ref, w2_ref, b1_ref, y2_ref):
    dis = dis_ref[...]
    acc = jnp.concatenate([acc_ref[0], acc_ref[1]], axis=1)
    y1 = jnp.concatenate([y1_ref[0], y1_ref[1]], axis=1)
    x1 = jnp.maximum(dis * (acc + y1) + b1_ref[...], 0.0)
    y2 = jnp.dot(x1, w2_ref[...], preferred_element_type=jnp.float32) * dis
    y2_ref[0] = y2[:, :FH]
    y2_ref[1] = y2[:, FH:]


def _k4(acc1, y1, dis2d, W2, b1):
    return pl.pallas_call(
        _k4_body,
        grid=(_NG,),
        in_specs=[
            pl.BlockSpec((NSC, _RB, FH), lambda i: (0, i, 0)),
            pl.BlockSpec((NSC, _RB, FH), lambda i: (0, i, 0)),
            pl.BlockSpec((_RB, 1), lambda i: (i, 0)),
            pl.BlockSpec((FD, FD), lambda i: (0, 0)),
            pl.BlockSpec((1, FD), lambda i: (0, 0)),
        ],
        out_specs=pl.BlockSpec((NSC, _RB, FH), lambda i: (0, i, 0)),
        out_shape=jax.ShapeDtypeStruct((NSC, NPAD, FH), jnp.float32),
    )(acc1, y1, dis2d, W2, b1)


def _k6_body(part_ref, wm1_ref, bm1_ref, g1_ref, be1_ref,
             wm2_ref, bm2_ref, g2_ref, be2_ref, wm3_ref, bm3_ref, out_ref):
    m = jnp.max(part_ref[...], axis=1)
    x = jnp.concatenate([m[0], m[1]], axis=1)

    def bn_relu(h, g, be):
        m = jnp.mean(h, axis=0, keepdims=True)
        v = jnp.mean((h - m) ** 2, axis=0, keepdims=True)
        return jnp.maximum(g * (h - m) / jnp.sqrt(v + 1e-5) + be, 0.0)

    h = jnp.dot(x, wm1_ref[...], preferred_element_type=jnp.float32) + bm1_ref[...]
    h = bn_relu(h, g1_ref[...], be1_ref[...])
    h = jnp.dot(h, wm2_ref[...], preferred_element_type=jnp.float32) + bm2_ref[...]
    h = bn_relu(h, g2_ref[...], be2_ref[...])
    out_ref[...] = jnp.dot(h, wm3_ref[...], preferred_element_type=jnp.float32) + bm3_ref[...]


def _k6(part, Wm1, bm1, g1, be1, Wm2, bm2, g2, be2, Wm3, bm3):
    return pl.pallas_call(
        _k6_body,
        out_shape=jax.ShapeDtypeStruct((BB, 10), jnp.float32),
    )(part, Wm1, bm1, g1, be1, Wm2, bm2, g2, be2, Wm3, bm3)


def kernel(pos, edge_index, edge_attr, batch,
           W1, b1, W2, b2, Wm1, bm1, g1, be1, Wm2, bm2, g2, be2, Wm3, bm3):
    pe = EPAD - EE
    zpad = jnp.zeros((pe,), jnp.int32)
    rowp = jnp.concatenate([edge_index[0], zpad]).reshape(ER, KC)
    colp = jnp.concatenate([edge_index[1], zpad]).reshape(ER, KC)
    wp = jnp.concatenate([lax.bitcast_convert_type(edge_attr, jnp.int32),
                          zpad]).reshape(ER, KC)
    epack = jnp.stack([rowp, colp, wp], axis=1)
    posp = jnp.concatenate([pos, jnp.zeros((NPAD - NN, 3), jnp.float32)])
    batp = jnp.concatenate([batch, jnp.full((NPAD - NN,), BB - 1, jnp.int32)])

    deg = _deg_call(epack)
    dis2d, y1 = _k2(posp, deg.reshape(NSC, NPAD, 1), W1)
    acc1 = _conv_call(y1.reshape(NSC * NPAD, FH), epack)
    y2 = _k4(acc1, y1, dis2d, W2, b1.reshape(1, FD))
    part = _conv_segmax_call(y2.reshape(NSC * NPAD, FH), epack,
                             dis2d.reshape(NPAD), batp, b2)
    return _k6(part, Wm1, bm1.reshape(1, FD), g1.reshape(1, FD), be1.reshape(1, FD),
               Wm2, bm2.reshape(1, FD), g2.reshape(1, FD), be2.reshape(1, FD),
               Wm3, bm3.reshape(1, 10))


# speedup vs baseline: 21.3297x; 1.0182x over previous
import jax
import jax.numpy as jnp
from jax import lax
from jax.experimental import pallas as pl
from jax.experimental.pallas import tpu as pltpu
from jax.experimental.pallas import tpu_sc as plsc

NN = 50000
EE = 800000
BB = 64
FD = 64

NSC = 2
NTL = 16

NPAD = 50176
FH = FD // NSC
RT = NPAD // NTL

KC = 112
SUB = 8
OUTER = 56
EPT = OUTER * SUB * KC
EPAD = EPT * NTL
ER = EPAD // KC
ERT = ER // NTL

RC = 112

_mesh = plsc.VectorSubcoreMesh(core_axis_name="c", subcore_axis_name="s")
_sc_params = pltpu.CompilerParams(use_tc_tiling_on_sc=False, needs_layout_passes=False)


def _zero_rows(buf, rows):
    z = jnp.zeros((16,), jnp.float32)

    def body(k, _):
        for q in range(FH // 16):
            buf[k, pl.ds(q * 16, 16)] = z
        return _

    lax.fori_loop(0, rows, body, None)


def _zero_shared2(zbuf, sh, s):
    base = s * RT
    for i in range(RT // KC):
        pltpu.sync_copy(zbuf.at[pl.ds(0, KC)], sh.at[pl.ds(base + i * KC, KC)])


def _edge_scatter_pass(y_hbm, epk_hbm, acc_sh, epk, rb0, rb1, rb2,
                       semi0, semi1, sg0, sg1, sg2, ss0, ss1, ss2, c, s):
    rbufs = (rb0, rb1, rb2)
    semg = (sg0, sg1, sg2)
    sems = (ss0, ss1, ss2)

    def process(b):
        off = c * NPAD
        for j in range(SUB):
            for q in range(KC // 16):
                sl = pl.ds(q * 16, 16)
                epk[b, j, 0, sl] = epk[b, j, 0, sl] + off
        dg = [None, None, None]
        ds_ = [None, None, None]
        dg[0] = pltpu.async_copy(y_hbm.at[epk.at[b, 0, 0]], rbufs[0], semg[0])
        for j in range(SUB):
            p = j % 3
            if j < SUB - 1:
                pn = (j + 1) % 3
                if ds_[pn] is not None:
                    ds_[pn].wait()
                dg[pn] = pltpu.async_copy(
                    y_hbm.at[epk.at[b, j + 1, 0]], rbufs[pn], semg[pn])
            dg[p].wait()

            def mul(g, _):
                wv = plsc.bitcast(epk[b, j, 2, pl.ds(g * 16, 16)], jnp.float32)
                kb = g * 16
                for u in range(16):
                    wk = wv[u]
                    for q in range(FH // 16):
                        sl = pl.ds(q * 16, 16)
                        rbufs[p][kb + u, sl] = rbufs[p][kb + u, sl] * wk
                return _

            lax.fori_loop(0, KC // 16, mul, None)
            ds_[p] = pltpu.async_copy(
                rbufs[p], acc_sh.at[epk.at[b, j, 1]], sems[p], add=True)
        for p in range(3):
            ds_[p].wait()

    def outer(i, _):
        r0 = s * ERT + (2 * i) * SUB
        da = pltpu.async_copy(epk_hbm.at[pl.ds(r0, SUB)], epk.at[0], semi0)
        db = pltpu.async_copy(epk_hbm.at[pl.ds(r0 + SUB, SUB)], epk.at[1], semi1)
        da.wait()
        process(0)
        db.wait()
        process(1)
        return _

    lax.fori_loop(0, OUTER // 2, outer, None)


DPAD = NPAD + 128
DZT = DPAD // NTL
DWT = NPAD // NTL
DOUT = OUTER // 2


def _deg_body(epk_hbm, deg_hbm, deg_sh, epk, wf, zbuf, semi, sems):
    c = lax.axis_index("c")
    s = lax.axis_index("s")
    z = jnp.zeros((16,), jnp.float32)
    for q in range(KC // 16):
        zbuf[pl.ds(q * 16, 16)] = z
    base = s * DZT
    nfull = DZT // KC
    rem = DZT - nfull * KC
    for i in range(nfull):
        pltpu.sync_copy(zbuf, deg_sh.at[pl.ds(base + i * KC, KC)])
    pltpu.sync_copy(zbuf.at[pl.ds(0, rem)], deg_sh.at[pl.ds(base + nfull * KC, rem)])
    plsc.subcore_barrier()

    tbase = c * (ER // 2) + s * (ERT // 2)

    def prep(ch, b):
        pltpu.async_copy(epk_hbm.at[pl.ds(tbase + ch * SUB, SUB)],
                         epk.at[b], semi).wait()
        for j in range(SUB):
            for q in range(KC // 16):
                sl = pl.ds(q * 16, 16)
                wf[b, j, sl] = plsc.bitcast(epk[b, j, 2, sl], jnp.float32)

    def fire(b):
        return [pltpu.async_copy(wf.at[b, j], deg_sh.at[epk.at[b, j, 1]],
                                 sems, add=True) for j in range(SUB)]

    def drain(ds_):
        for d in ds_:
            d.wait()

    def outer(i, _):
        ch = i * 4
        prep(ch, 0)
        dsA = fire(0)
        prep(ch + 1, 1)
        drain(dsA)
        dsB = fire(1)
        prep(ch + 2, 0)
        drain(dsB)
        dsC = fire(0)
        prep(ch + 3, 1)
        drain(dsC)
        dsD = fire(1)
        drain(dsD)
        return _

    lax.fori_loop(0, DOUT // 4, outer, None)
    plsc.subcore_barrier()
    for i in range(DWT // KC):
        pltpu.sync_copy(deg_sh.at[pl.ds(s * DWT + i * KC, KC)], zbuf)
        pltpu.sync_copy(zbuf, deg_hbm.at[c, pl.ds(s * DWT + i * KC, KC)])


_deg_call = pl.kernel(
    _deg_body,
    out_type=jax.ShapeDtypeStruct((NSC, NPAD), jnp.float32),
    mesh=_mesh,
    compiler_params=_sc_params,
    scratch_types=[
        pltpu.VMEM_SHARED((DPAD,), jnp.float32),
        pltpu.VMEM((2, SUB, 3, KC), jnp.int32),
        pltpu.VMEM((2, SUB, KC), jnp.float32),
        pltpu.VMEM((KC,), jnp.float32),
        pltpu.SemaphoreType.DMA,
        pltpu.SemaphoreType.DMA,
    ],
)


def _conv_body(y_hbm, epk_hbm, acc_hbm, acc_sh, epk, rb0, rb1, rb2,
               semi0, semi1, sg0, sg1, sg2, ss0, ss1, ss2):
    c = lax.axis_index("c")
    s = lax.axis_index("s")
    _zero_rows(rb0, KC)
    _zero_shared2(rb0, acc_sh, s)
    plsc.subcore_barrier()
    _edge_scatter_pass(y_hbm, epk_hbm, acc_sh, epk, rb0, rb1, rb2,
                       semi0, semi1, sg0, sg1, sg2, ss0, ss1, ss2, c, s)
    plsc.subcore_barrier()
    pltpu.sync_copy(acc_sh.at[pl.ds(s * RT, RT)],
                    acc_hbm.at[c, pl.ds(s * RT, RT)])


_conv_call = pl.kernel(
    _conv_body,
    out_type=jax.ShapeDtypeStruct((NSC, NPAD, FH), jnp.float32),
    mesh=_mesh,
    compiler_params=_sc_params,
    scratch_types=[
        pltpu.VMEM_SHARED((NPAD, FH), jnp.float32),
        pltpu.VMEM((2, SUB, 3, KC), jnp.int32),
        pltpu.VMEM((KC, FH), jnp.float32),
        pltpu.VMEM((KC, FH), jnp.float32),
        pltpu.VMEM((KC, FH), jnp.float32),
        pltpu.SemaphoreType.DMA,
        pltpu.SemaphoreType.DMA,
        pltpu.SemaphoreType.DMA,
        pltpu.SemaphoreType.DMA,
        pltpu.SemaphoreType.DMA,
        pltpu.SemaphoreType.DMA,
        pltpu.SemaphoreType.DMA,
        pltpu.SemaphoreType.DMA,
    ],
)


def _conv_segmax_body(y_hbm, epk_hbm, dis_hbm, bat_hbm, b2_hbm,
                      part_hbm, acc_sh, epk, rb0, rb1, rb2,
                      dbuf, bbuf, b2buf, segbuf,
                      semi0, semi1, sg0, sg1, sg2, ss0, ss1, ss2):
    c = lax.axis_index("c")
    s = lax.axis_index("s")
    _zero_rows(rb0, KC)
    _zero_shared2(rb0, acc_sh, s)
    plsc.subcore_barrier()
    _edge_scatter_pass(y_hbm, epk_hbm, acc_sh, epk, rb0, rb1, rb2,
                       semi0, semi1, sg0, sg1, sg2, ss0, ss1, ss2, c, s)
    plsc.subcore_barrier()

    ninf = jnp.full((16,), -jnp.inf, jnp.float32)

    def seg_init(k, _):
        for q in range(FH // 16):
            segbuf[k, pl.ds(q * 16, 16)] = ninf
        return _

    lax.fori_loop(0, BB, seg_init, None)

    pltpu.sync_copy(b2_hbm, b2buf)
    gstart = s * RT

    def seg_chunk(i, _):
        g0 = gstart + i * RC
        pltpu.sync_copy(acc_sh.at[pl.ds(g0, RC)], rb0.at[pl.ds(0, RC)])
        pltpu.sync_copy(y_hbm.at[pl.ds(c * NPAD + g0, RC)], rb1.at[pl.ds(0, RC)])
        pltpu.sync_copy(dis_hbm.at[pl.ds(g0, RC)], dbuf)
        pltpu.sync_copy(bat_hbm.at[pl.ds(g0, RC)], bbuf)

        def seg_grp(g, _):
            dv = dbuf[pl.ds(g * 16, 16)]
            bv = bbuf[pl.ds(g * 16, 16)]
            kb = g * 16
            gbase = g0 + kb
            for u in range(16):
                dk = dv[u]
                bid = bv[u]
                pen = jnp.where(gbase + u < NN, 0.0, -jnp.inf).astype(jnp.float32)
                for q in range(FH // 16):
                    sl = pl.ds(q * 16, 16)
                    b2v = b2buf[pl.ds(c * FH + q * 16, 16)]
                    v = dk * (rb0[kb + u, sl] + rb1[kb + u, sl]) + b2v + pen
                    segbuf[bid, sl] = jnp.maximum(segbuf[bid, sl], v)
            return _

        lax.fori_loop(0, RC // 16, seg_grp, None)
        return _

    lax.fori_loop(0, RT // RC, seg_chunk, None)
    pltpu.sync_copy(segbuf, part_hbm.at[c, s])


_conv_segmax_call = pl.kernel(
    _conv_segmax_body,
    out_type=jax.ShapeDtypeStruct((NSC, NTL, BB, FH), jnp.float32),
    mesh=_mesh,
    compiler_params=_sc_params,
    scratch_types=[
        pltpu.VMEM_SHARED((NPAD, FH), jnp.float32),
        pltpu.VMEM((2, SUB, 3, KC), jnp.int32),
        pltpu.VMEM((KC, FH), jnp.float32),
        pltpu.VMEM((KC, FH), jnp.float32),
        pltpu.VMEM((KC, FH), jnp.float32),
        pltpu.VMEM((RC,), jnp.float32),
        pltpu.VMEM((RC,), jnp.int32),
        pltpu.VMEM((FD,), jnp.float32),
        pltpu.VMEM((BB, FH), jnp.float32),
        pltpu.SemaphoreType.DMA,
        pltpu.SemaphoreType.DMA,
        pltpu.SemaphoreType.DMA,
        pltpu.SemaphoreType.DMA,
        pltpu.SemaphoreType.DMA,
        pltpu.SemaphoreType.DMA,
        pltpu.SemaphoreType.DMA,
        pltpu.SemaphoreType.DMA,
    ],
)


_RB = 1024
_NG = NPAD // _RB


def _k2_body(pos_ref, deg_ref, w1_ref, dis_ref, y1_ref):
    deg = deg_ref[0] + deg_ref[1]
    dis = lax.rsqrt(deg + 1.0)
    dis_ref[...] = dis
    xl = jnp.dot(pos_ref[...], w1_ref[...], preferred_element_type=jnp.float32)
    y = xl * dis
    y1_ref[0] = y[:, :FH]
    y1_ref[1] = y[:, FH:]


def _k2(posp, deg3d, W1):
    return pl.pallas_call(
        _k2_body,
        grid=(_NG,),
        in_specs=[
            pl.BlockSpec((_RB, 3), lambda i: (i, 0)),
            pl.BlockSpec((NSC, _RB, 1), lambda i: (0, i, 0)),
            pl.BlockSpec((3, FD), lambda i: (0, 0)),
        ],
        out_specs=[
            pl.BlockSpec((_RB, 1), lambda i: (i, 0)),
            pl.BlockSpec((NSC, _RB, FH), lambda i: (0, i, 0)),
        ],
        out_shape=[
            jax.ShapeDtypeStruct((NPAD, 1), jnp.float32),
            jax.ShapeDtypeStruct((NSC, NPAD, FH), jnp.float32),
        ],
    )(posp, deg3d, W1)


def _k4_body(acc_ref, y1_ref, dis_ref, w2_ref, b1_ref, y2_ref):
    dis = dis_ref[...]
    acc = jnp.concatenate([acc_ref[0], acc_ref[1]], axis=1)
    y1 = jnp.concatenate([y1_ref[0], y1_ref[1]], axis=1)
    x1 = jnp.maximum(dis * (acc + y1) + b1_ref[...], 0.0)
    y2 = jnp.dot(x1, w2_ref[...], preferred_element_type=jnp.float32) * dis
    y2_ref[0] = y2[:, :FH]
    y2_ref[1] = y2[:, FH:]


def _k4(acc1, y1, dis2d, W2, b1):
    return pl.pallas_call(
        _k4_body,
        grid=(_NG,),
        in_specs=[
            pl.BlockSpec((NSC, _RB, FH), lambda i: (0, i, 0)),
            pl.BlockSpec((NSC, _RB, FH), lambda i: (0, i, 0)),
            pl.BlockSpec((_RB, 1), lambda i: (i, 0)),
            pl.BlockSpec((FD, FD), lambda i: (0, 0)),
            pl.BlockSpec((1, FD), lambda i: (0, 0)),
        ],
        out_specs=pl.BlockSpec((NSC, _RB, FH), lambda i: (0, i, 0)),
        out_shape=jax.ShapeDtypeStruct((NSC, NPAD, FH), jnp.float32),
    )(acc1, y1, dis2d, W2, b1)


def _k6_body(part_ref, wm1_ref, bm1_ref, g1_ref, be1_ref,
             wm2_ref, bm2_ref, g2_ref, be2_ref, wm3_ref, bm3_ref, out_ref):
    m = jnp.max(part_ref[...], axis=1)
    x = jnp.concatenate([m[0], m[1]], axis=1)

    def bn_relu(h, g, be):
        m = jnp.mean(h, axis=0, keepdims=True)
        v = jnp.mean((h - m) ** 2, axis=0, keepdims=True)
        return jnp.maximum(g * (h - m) / jnp.sqrt(v + 1e-5) + be, 0.0)

    h = jnp.dot(x, wm1_ref[...], preferred_element_type=jnp.float32) + bm1_ref[...]
    h = bn_relu(h, g1_ref[...], be1_ref[...])
    h = jnp.dot(h, wm2_ref[...], preferred_element_type=jnp.float32) + bm2_ref[...]
    h = bn_relu(h, g2_ref[...], be2_ref[...])
    out_ref[...] = jnp.dot(h, wm3_ref[...], preferred_element_type=jnp.float32) + bm3_ref[...]


def _k6(part, Wm1, bm1, g1, be1, Wm2, bm2, g2, be2, Wm3, bm3):
    return pl.pallas_call(
        _k6_body,
        out_shape=jax.ShapeDtypeStruct((BB, 10), jnp.float32),
    )(part, Wm1, bm1, g1, be1, Wm2, bm2, g2, be2, Wm3, bm3)


def kernel(pos, edge_index, edge_attr, batch,
           W1, b1, W2, b2, Wm1, bm1, g1, be1, Wm2, bm2, g2, be2, Wm3, bm3):
    pe = EPAD - EE
    zpad = jnp.zeros((pe,), jnp.int32)
    rowp = jnp.concatenate([edge_index[0], zpad]).reshape(ER, KC)
    colp = jnp.concatenate([edge_index[1], zpad]).reshape(ER, KC)
    wp = jnp.concatenate([lax.bitcast_convert_type(edge_attr, jnp.int32),
                          zpad]).reshape(ER, KC)
    epack = jnp.stack([rowp, colp, wp], axis=1)
    posp = jnp.concatenate([pos, jnp.zeros((NPAD - NN, 3), jnp.float32)])
    batp = jnp.concatenate([batch, jnp.full((NPAD - NN,), BB - 1, jnp.int32)])

    deg = _deg_call(epack)
    dis2d, y1 = _k2(posp, deg.reshape(NSC, NPAD, 1), W1)
    acc1 = _conv_call(y1.reshape(NSC * NPAD, FH), epack)
    y2 = _k4(acc1, y1, dis2d, W2, b1.reshape(1, FD))
    part = _conv_segmax_call(y2.reshape(NSC * NPAD, FH), epack,
                             dis2d.reshape(NPAD), batp, b2)
    return _k6(part, Wm1, bm1.reshape(1, FD), g1.reshape(1, FD), be1.reshape(1, FD),
               Wm2, bm2.reshape(1, FD), g2.reshape(1, FD), be2.reshape(1, FD),
               Wm3, bm3.reshape(1, 10))
